# baseline (speedup 1.0000x reference)
import functools
import jax
import jax.numpy as jnp
from jax import lax
from jax.experimental import pallas as pl
from jax.experimental.pallas import tpu as pltpu
from jax.experimental.pallas import tpu_sc as plsc

VOCAB = 100000
H = 64
BATCH = 4096
C = 50
T = 50
NC = 2
NS = 16
NW = NC * NS
BPW = BATCH // NW
NQ = H // 16
TP = 64
CP = 56

_mesh = plsc.VectorSubcoreMesh(core_axis_name="c", subcore_axis_name="s")


@functools.partial(
    pl.kernel,
    mesh=_mesh,
    compiler_params=pltpu.CompilerParams(use_tc_tiling_on_sc=False),
    out_type=jax.ShapeDtypeStruct((BATCH, H), jnp.float32),
    scratch_types=[
        pltpu.VMEM((BPW, CP), jnp.int32),
        pltpu.VMEM((CP, H), jnp.float32),
        pltpu.VMEM((CP, H), jnp.float32),
        pltpu.VMEM((BPW, H), jnp.float32),
        pltpu.SemaphoreType.DMA,
        pltpu.SemaphoreType.DMA,
    ],
)
def _con_sc(ctx_hbm, in_hbm, con_hbm, ctx_idx, crows0, crows1, cb,
            sem0, sem1):
    wid = lax.axis_index("s") * NC + lax.axis_index("c")
    base = wid * BPW
    pltpu.sync_copy(ctx_hbm.at[pl.ds(base, BPW)], ctx_idx)

    def fire(b, crows, sem):
        pltpu.async_copy(in_hbm.at[ctx_idx.at[b, pl.ds(0, 32)]],
                         crows.at[pl.ds(0, 32)], sem)
        pltpu.async_copy(in_hbm.at[ctx_idx.at[b, pl.ds(32, 24)]],
                         crows.at[pl.ds(32, 24)], sem)

    def drain(crows, sem):
        pltpu.make_async_copy(in_hbm.at[pl.ds(0, CP)], crows, sem).wait()

    def compute(b, crows):
        acc = [jnp.zeros((16,), jnp.float32) for _ in range(2 * NQ)]
        for c in range(C):
            for q in range(NQ):
                a = (c % 2) * NQ + q
                acc[a] = acc[a] + crows[c, pl.ds(q * 16, 16)]
        for q in range(NQ):
            cb[b, pl.ds(q * 16, 16)] = acc[q] + acc[NQ + q]

    fire(0, crows0, sem0)

    def body(i, carry):
        b0 = 2 * i
        fire(b0 + 1, crows1, sem1)
        drain(crows0, sem0)
        compute(b0, crows0)
        fire(jnp.minimum(b0 + 2, BPW - 1), crows0, sem0)
        drain(crows1, sem1)
        compute(b0 + 1, crows1)
        return carry

    lax.fori_loop(0, BPW // 2, body, 0)
    drain(crows0, sem0)
    pltpu.sync_copy(cb, con_hbm.at[pl.ds(base, BPW)])


@functools.partial(
    pl.kernel,
    mesh=_mesh,
    compiler_params=pltpu.CompilerParams(use_tc_tiling_on_sc=False),
    out_type=jax.ShapeDtypeStruct((BATCH, TP), jnp.float32),
    scratch_types=[
        pltpu.VMEM((BPW, CP), jnp.int32),
        pltpu.VMEM((BPW, H), jnp.float32),
        pltpu.VMEM((CP, H), jnp.float32),
        pltpu.VMEM((CP, H), jnp.float32),
        pltpu.VMEM((BPW, TP), jnp.float32),
        pltpu.SemaphoreType.DMA,
        pltpu.SemaphoreType.DMA,
    ],
)
def _score_sc(t_hbm, out_hbm, con_hbm, y_hbm,
              t_idx, cb, trows0, trows1, yb, sem0, sem1):
    wid = lax.axis_index("s") * NC + lax.axis_index("c")
    base = wid * BPW
    pltpu.sync_copy(t_hbm.at[pl.ds(base, BPW)], t_idx)
    pltpu.sync_copy(con_hbm.at[pl.ds(base, BPW)], cb)
    lane = lax.iota(jnp.int32, 16)
    perms = [(lane + (1 << k)) % 16 for k in range(4)]

    def hsum(v):
        for pidx in perms:
            v = v + jnp.take(v, pidx, mode="wrap")
        return v

    def fire(b, trows, sem):
        pltpu.async_copy(out_hbm.at[t_idx.at[b, pl.ds(0, 32)]],
                         trows.at[pl.ds(0, 32)], sem)
        pltpu.async_copy(out_hbm.at[t_idx.at[b, pl.ds(32, 24)]],
                         trows.at[pl.ds(32, 24)], sem)

    def drain(trows, sem):
        pltpu.make_async_copy(out_hbm.at[pl.ds(0, CP)], trows, sem).wait()

    def compute(b, trows):
        con = [cb[b, pl.ds(q * 16, 16)] for q in range(NQ)]
        for g in range(TP // 16):
            yv = jnp.zeros((16,), jnp.float32)
            for jj in range(16):
                j = g * 16 + jj
                if j >= T:
                    continue
                p = trows[j, pl.ds(0, 16)] * con[0]
                for q in range(1, NQ):
                    p = p + trows[j, pl.ds(q * 16, 16)] * con[q]
                yv = jnp.where(lane == jj, hsum(p), yv)
            yb[b, pl.ds(g * 16, 16)] = yv

    fire(0, trows0, sem0)

    def body(i, carry):
        b0 = 2 * i
        fire(b0 + 1, trows1, sem1)
        drain(trows0, sem0)
        compute(b0, trows0)
        fire(jnp.minimum(b0 + 2, BPW - 1), trows0, sem0)
        drain(trows1, sem1)
        compute(b0 + 1, trows1)
        return carry

    lax.fori_loop(0, BPW // 2, body, 0)
    drain(trows0, sem0)
    pltpu.sync_copy(yb, y_hbm.at[pl.ds(base, BPW)])


def kernel(contexts, t, in_emb, out_emb):
    ctx56 = jnp.pad(contexts.astype(jnp.int32), ((0, 0), (0, CP - C)))
    t56 = jnp.pad(t.astype(jnp.int32), ((0, 0), (0, CP - T)))
    con = _con_sc(ctx56, in_emb)
    y = _score_sc(t56, out_emb, con)
    return y[:, :T].reshape(BATCH, 1, T)


# speedup vs baseline: 5.0533x; 5.0533x over previous
import functools
import jax
import jax.numpy as jnp
from jax import lax
from jax.experimental import pallas as pl
from jax.experimental.pallas import tpu as pltpu
from jax.experimental.pallas import tpu_sc as plsc

VOCAB = 100000
H = 64
BATCH = 4096
C = 50
T = 50
NC = 2
NS = 16
NW = NC * NS
BPW = BATCH // NW
NQ = H // 16
TP = 64

_mesh = plsc.VectorSubcoreMesh(core_axis_name="c", subcore_axis_name="s")


@functools.partial(
    pl.kernel,
    mesh=_mesh,
    compiler_params=pltpu.CompilerParams(use_tc_tiling_on_sc=False),
    out_type=jax.ShapeDtypeStruct((BATCH, TP), jnp.float32),
    scratch_types=[
        pltpu.VMEM((BPW, C), jnp.int32),
        pltpu.VMEM((BPW, T), jnp.int32),
        pltpu.VMEM((C, H), jnp.float32),
        pltpu.VMEM((C, H), jnp.float32),
        pltpu.VMEM((T, H), jnp.float32),
        pltpu.VMEM((T, H), jnp.float32),
        pltpu.VMEM((BPW, TP), jnp.float32),
        pltpu.SemaphoreType.DMA,
        pltpu.SemaphoreType.DMA,
    ],
)
def _cbow_sc(ctx_hbm, t_hbm, in_hbm, out_hbm, y_hbm,
             ctx_idx, t_idx, crows0, crows1, trows0, trows1, yb,
             sem0, sem1):
    wid = lax.axis_index("s") * NC + lax.axis_index("c")
    base = wid * BPW
    pltpu.sync_copy(ctx_hbm.at[pl.ds(base, BPW)], ctx_idx)
    pltpu.sync_copy(t_hbm.at[pl.ds(base, BPW)], t_idx)
    lane = lax.iota(jnp.int32, 16)
    perms = [(lane + (1 << k)) % 16 for k in range(4)]

    def hsum(v):
        for pidx in perms:
            v = v + jnp.take(v, pidx, mode="wrap")
        return v

    def fire(b, crows, trows, sem):
        pltpu.async_copy(in_hbm.at[ctx_idx.at[b]], crows, sem)
        pltpu.async_copy(out_hbm.at[t_idx.at[b]], trows, sem)

    def drain(crows, trows, sem):
        pltpu.make_async_copy(in_hbm.at[pl.ds(0, C)], crows, sem).wait()
        pltpu.make_async_copy(in_hbm.at[pl.ds(0, T)], trows, sem).wait()

    def compute(b, crows, trows):
        acc = [jnp.zeros((16,), jnp.float32) for _ in range(2 * NQ)]
        for c in range(C):
            for q in range(NQ):
                a = (c % 2) * NQ + q
                acc[a] = acc[a] + crows[c, pl.ds(q * 16, 16)]
        con = [acc[q] + acc[NQ + q] for q in range(NQ)]
        for g in range(TP // 16):
            yv = jnp.zeros((16,), jnp.float32)
            for jj in range(16):
                j = g * 16 + jj
                if j >= T:
                    continue
                p = trows[j, pl.ds(0, 16)] * con[0]
                for q in range(1, NQ):
                    p = p + trows[j, pl.ds(q * 16, 16)] * con[q]
                yv = jnp.where(lane == jj, hsum(p), yv)
            yb[b, pl.ds(g * 16, 16)] = yv

    fire(0, crows0, trows0, sem0)

    def body(i, carry):
        b0 = 2 * i
        fire(b0 + 1, crows1, trows1, sem1)
        drain(crows0, trows0, sem0)
        compute(b0, crows0, trows0)
        fire(jnp.minimum(b0 + 2, BPW - 1), crows0, trows0, sem0)
        drain(crows1, trows1, sem1)
        compute(b0 + 1, crows1, trows1)
        return carry

    lax.fori_loop(0, BPW // 2, body, 0)
    drain(crows0, trows0, sem0)
    pltpu.sync_copy(yb, y_hbm.at[pl.ds(base, BPW)])


def kernel(contexts, t, in_emb, out_emb):
    y = _cbow_sc(contexts.astype(jnp.int32), t.astype(jnp.int32),
                 in_emb, out_emb)
    return y[:, :T].reshape(BATCH, 1, T)
